# baseline (device time: 269013 ns/iter reference)
import jax
import jax.numpy as jnp
from jax import lax
from jax.experimental import pallas as pl
from jax.experimental.pallas import tpu as pltpu

N_DEV = 32
CW_HOPS = 16
CCW_HOPS = 15
VOUT_SLOTS = 4


def _ring_tables():
    logical = []
    for z in range(4):
        for y in range(4):
            row = [(0, y, z), (1, y, z)]
            if y % 2:
                row.reverse()
            logical += row
    path = []
    for z in range(4):
        ys = range(4) if z % 2 == 0 else range(3, -1, -1)
        path += [(y, z) for y in ys]
    ring = [(0, y, z) for (y, z) in path]
    ring += [(1, y, z) for (y, z) in reversed(path)]
    c2l = {c: i for i, c in enumerate(logical)}
    pi = [c2l[c] for c in ring]
    pinv = [0] * N_DEV
    for p, l in enumerate(pi):
        pinv[l] = p
    return pi, pinv


_PI, _PINV = _ring_tables()


def kernel(A, B):
    m_per, k = A.shape
    n = B.shape[1]
    A = A.astype(jnp.bfloat16)
    B = B.astype(jnp.bfloat16)
    pi = jnp.asarray(_PI, dtype=jnp.int32)
    pinv = jnp.asarray(_PINV, dtype=jnp.int32)

    def body(a_ref, b_ref, pi_ref, pinv_ref, out_ref, comm, vout,
             recv_sems, send_cw_sems, send_ccw_sems, copy_sems):
        my = lax.axis_index("i")
        p = pinv_ref[my]
        right = pi_ref[lax.rem(p + 1, N_DEV)]
        left = pi_ref[lax.rem(p + N_DEV - 1, N_DEV)]

        def rpos(delta):
            return lax.rem(p + N_DEV + delta, N_DEV)

        barrier_sem = pltpu.get_barrier_semaphore()
        for nbr in (left, right):
            pl.semaphore_signal(barrier_sem, inc=1, device_id=(nbr,),
                                device_id_type=pl.DeviceIdType.MESH)
        pl.semaphore_wait(barrier_sem, 2)

        comm[pl.ds(p, 1)] = a_ref[:, :].reshape(1, m_per, k)

        pending_copies = []

        def compute_and_store(rp, j):
            s = j % VOUT_SLOTS
            if j >= VOUT_SLOTS:
                pending_copies[j - VOUT_SLOTS][0].wait()
            tile = jax.lax.dot_general(
                comm[rp], b_ref[:, :],
                dimension_numbers=(((1,), (0,)), ((), ())),
                preferred_element_type=jnp.float32,
            )
            vout[s] = tile.astype(jnp.bfloat16)
            origin = pi_ref[rp]
            cp = pltpu.make_async_copy(
                vout.at[s],
                out_ref.at[pl.ds(origin * m_per, m_per), :],
                copy_sems.at[j % N_DEV],
            )
            cp.start()
            pending_copies.append((cp, j))

        def recv_wait(rp):
            pltpu.make_async_remote_copy(
                src_ref=comm.at[rp], dst_ref=comm.at[rp],
                send_sem=send_cw_sems.at[0], recv_sem=recv_sems.at[rp],
                device_id=(right,), device_id_type=pl.DeviceIdType.MESH,
            ).wait_recv()

        sends = []
        ready = [p]
        j = 0
        for h in range(CW_HOPS):
            rp_s = rpos(-h)
            cw = pltpu.make_async_remote_copy(
                src_ref=comm.at[rp_s],
                dst_ref=comm.at[rp_s],
                send_sem=send_cw_sems.at[h],
                recv_sem=recv_sems.at[rp_s],
                device_id=(right,),
                device_id_type=pl.DeviceIdType.MESH,
            )
            cw.start()
            sends.append(cw)

            if h < CCW_HOPS:
                rp_s2 = rpos(h)
                ccw = pltpu.make_async_remote_copy(
                    src_ref=comm.at[rp_s2],
                    dst_ref=comm.at[rp_s2],
                    send_sem=send_ccw_sems.at[h],
                    recv_sem=recv_sems.at[rp_s2],
                    device_id=(left,),
                    device_id_type=pl.DeviceIdType.MESH,
                )
                ccw.start()
                sends.append(ccw)

            while len(ready) > 2:
                compute_and_store(ready.pop(0), j)
                j += 1

            rp_r = rpos(-h - 1)
            recv_wait(rp_r)
            ready.append(rp_r)
            if h < CCW_HOPS:
                rp_r2 = rpos(h + 1)
                recv_wait(rp_r2)
                ready.append(rp_r2)

        for rp in ready:
            compute_and_store(rp, j)
            j += 1

        for s in sends:
            s.wait_send()
        for cp, jj in pending_copies[-VOUT_SLOTS:]:
            cp.wait()

    out_shape = jax.ShapeDtypeStruct((N_DEV * m_per, n), jnp.bfloat16)
    return pl.pallas_call(
        body,
        out_shape=out_shape,
        in_specs=[
            pl.BlockSpec(memory_space=pltpu.VMEM),
            pl.BlockSpec(memory_space=pltpu.VMEM),
            pl.BlockSpec(memory_space=pltpu.SMEM),
            pl.BlockSpec(memory_space=pltpu.SMEM),
        ],
        out_specs=pl.BlockSpec(memory_space=pl.ANY),
        scratch_shapes=[
            pltpu.VMEM((N_DEV, m_per, k), jnp.bfloat16),
            pltpu.VMEM((VOUT_SLOTS, m_per, n), jnp.bfloat16),
            pltpu.SemaphoreType.DMA((N_DEV,)),
            pltpu.SemaphoreType.DMA((CW_HOPS,)),
            pltpu.SemaphoreType.DMA((CCW_HOPS,)),
            pltpu.SemaphoreType.DMA((N_DEV,)),
        ],
        compiler_params=pltpu.CompilerParams(
            collective_id=0,
            vmem_limit_bytes=100 * 1024 * 1024,
        ),
    )(A, B, pi, pinv)


# device time: 265111 ns/iter; 1.0147x vs baseline; 1.0147x over previous
import jax
import jax.numpy as jnp
from jax import lax
from jax.experimental import pallas as pl
from jax.experimental.pallas import tpu as pltpu

N_DEV = 32
CW_HOPS = 16
CCW_HOPS = 15
VOUT_SLOTS = 4


def _ring_tables():
    logical = []
    for z in range(4):
        for y in range(4):
            row = [(0, y, z), (1, y, z)]
            if y % 2:
                row.reverse()
            logical += row
    path = []
    for z in range(4):
        ys = range(4) if z % 2 == 0 else range(3, -1, -1)
        path += [(y, z) for y in ys]
    ring = [(0, y, z) for (y, z) in path]
    ring += [(1, y, z) for (y, z) in reversed(path)]
    c2l = {c: i for i, c in enumerate(logical)}
    pi = [c2l[c] for c in ring]
    pinv = [0] * N_DEV
    for p, l in enumerate(pi):
        pinv[l] = p
    return pi, pinv


_PI, _PINV = _ring_tables()


def kernel(A, B):
    m_per, k = A.shape
    n = B.shape[1]
    A = A.astype(jnp.bfloat16)
    B = B.astype(jnp.bfloat16)
    pi = jnp.asarray(_PI, dtype=jnp.int32)
    pinv = jnp.asarray(_PINV, dtype=jnp.int32)

    def body(a_ref, b_ref, pi_ref, pinv_ref, out_ref, comm, vout,
             recv_sems, send_cw_sems, send_ccw_sems, copy_sems):
        my = lax.axis_index("i")
        p = pinv_ref[my]
        right = pi_ref[lax.rem(p + 1, N_DEV)]
        left = pi_ref[lax.rem(p + N_DEV - 1, N_DEV)]

        def rpos(delta):
            return lax.rem(p + N_DEV + delta, N_DEV)

        barrier_sem = pltpu.get_barrier_semaphore()
        for nbr in (left, right):
            pl.semaphore_signal(barrier_sem, inc=1, device_id=(nbr,),
                                device_id_type=pl.DeviceIdType.MESH)
        pl.semaphore_wait(barrier_sem, 2)

        comm[pl.ds(p, 1)] = a_ref[:, :].reshape(1, m_per, k)

        pending_copies = []

        def compute_and_store(rp, j):
            s = j % VOUT_SLOTS
            if j >= VOUT_SLOTS:
                pending_copies[j - VOUT_SLOTS][0].wait()
            tile = jax.lax.dot_general(
                comm[rp], b_ref[:, :],
                dimension_numbers=(((1,), (0,)), ((), ())),
                preferred_element_type=jnp.float32,
            )
            vout[s] = tile.astype(jnp.bfloat16)
            origin = pi_ref[rp]
            cp = pltpu.make_async_copy(
                vout.at[s],
                out_ref.at[pl.ds(origin * m_per, m_per), :],
                copy_sems.at[j % N_DEV],
            )
            cp.start()
            pending_copies.append((cp, j))

        def recv_wait(rp):
            pltpu.make_async_remote_copy(
                src_ref=comm.at[rp], dst_ref=comm.at[rp],
                send_sem=send_cw_sems.at[0], recv_sem=recv_sems.at[rp],
                device_id=(right,), device_id_type=pl.DeviceIdType.MESH,
            ).wait_recv()

        sends = []

        def cw_send(hop):
            rp = rpos(-hop)
            d = pltpu.make_async_remote_copy(
                src_ref=comm.at[rp],
                dst_ref=comm.at[rp],
                send_sem=send_cw_sems.at[hop],
                recv_sem=recv_sems.at[rp],
                device_id=(right,),
                device_id_type=pl.DeviceIdType.MESH,
            )
            d.start()
            sends.append(d)

        def ccw_send(hop):
            rp = rpos(hop)
            d = pltpu.make_async_remote_copy(
                src_ref=comm.at[rp],
                dst_ref=comm.at[rp],
                send_sem=send_ccw_sems.at[hop],
                recv_sem=recv_sems.at[rp],
                device_id=(left,),
                device_id_type=pl.DeviceIdType.MESH,
            )
            d.start()
            sends.append(d)

        ready = [p]
        j = 0
        for h in range(CW_HOPS):
            cw_send(h)
            if h < CCW_HOPS:
                ccw_send(h)

            for rp in ready:
                compute_and_store(rp, j)
                j += 1
            ready = []

            rp_r = rpos(-h - 1)
            recv_wait(rp_r)
            ready.append(rp_r)
            if h < CCW_HOPS:
                rp_r2 = rpos(h + 1)
                recv_wait(rp_r2)
                ready.append(rp_r2)

        for rp in ready:
            compute_and_store(rp, j)
            j += 1

        for s in sends:
            s.wait_send()
        for cp, jj in pending_copies[-VOUT_SLOTS:]:
            cp.wait()

    out_shape = jax.ShapeDtypeStruct((N_DEV * m_per, n), jnp.bfloat16)
    return pl.pallas_call(
        body,
        out_shape=out_shape,
        in_specs=[
            pl.BlockSpec(memory_space=pltpu.VMEM),
            pl.BlockSpec(memory_space=pltpu.VMEM),
            pl.BlockSpec(memory_space=pltpu.SMEM),
            pl.BlockSpec(memory_space=pltpu.SMEM),
        ],
        out_specs=pl.BlockSpec(memory_space=pl.ANY),
        scratch_shapes=[
            pltpu.VMEM((N_DEV, m_per, k), jnp.bfloat16),
            pltpu.VMEM((VOUT_SLOTS, m_per, n), jnp.bfloat16),
            pltpu.SemaphoreType.DMA((N_DEV,)),
            pltpu.SemaphoreType.DMA((CW_HOPS,)),
            pltpu.SemaphoreType.DMA((CCW_HOPS,)),
            pltpu.SemaphoreType.DMA((N_DEV,)),
        ],
        compiler_params=pltpu.CompilerParams(
            collective_id=0,
            vmem_limit_bytes=100 * 1024 * 1024,
        ),
    )(A, B, pi, pinv)
